# baseline (speedup 1.0000x reference)
import functools

import jax
import jax.numpy as jnp
from jax import lax
from jax.experimental import pallas as pl
from jax.experimental.pallas import tpu as pltpu
from jax.experimental.pallas import tpu_sc as plsc

BSZ, T, C = 4, 4096, 1024
D = 1024
K = 2048
DS = 2
N = T // DS
M = BSZ * N

H = 4
MH = M // H
BM = 512
BK = 512

NCCH = 8
NC = 2
NS = 16
NW = NC * NS
RW = DS * MH // NW
CH = 32


def _lut_body(embed_ref, wout_ref, bout_ref, lut_ref, en_ref):
    e = embed_ref[:]
    lut_ref[:] = lax.dot_general(
        e, wout_ref[:], (((1,), (1,)), ((), ())),
        preferred_element_type=jnp.float32) + bout_ref[:]
    en_ref[:] = jnp.sum(e * e, axis=1)


def _dist_body(*refs):
    (x_refs, (wf_ref, bin_ref, embed_ref, en_ref),
     (idx_ref, loss_ref), (acc_ref,)) = (refs[:NCCH], refs[NCCH:NCCH + 4],
                                         refs[NCCH + 4:NCCH + 6],
                                         refs[NCCH + 6:])
    i = pl.program_id(0)
    xw = jnp.concatenate(
        [r[0, 0::2, :] for r in x_refs] + [r[0, 1::2, :] for r in x_refs],
        axis=1)
    z = lax.dot_general(
        xw, wf_ref[:], (((1,), (1,)), ((), ())),
        preferred_element_type=jnp.float32) + bin_ref[:]
    s = lax.dot_general(
        z, embed_ref[:], (((1,), (1,)), ((), ())),
        preferred_element_type=jnp.float32)
    zn = jnp.sum(z * z, axis=1, keepdims=True)
    dist = (zn - 2.0 * s) + en_ref[:][None, :]
    idx_ref[:] = jnp.argmin(dist, axis=1).astype(jnp.int32)
    mind = jnp.min(dist, axis=1)

    @pl.when(i == 0)
    def _():
        acc_ref[0] = 0.0

    acc_ref[0] += jnp.sum(mind)

    @pl.when(i == pl.num_programs(0) - 1)
    def _():
        loss_ref[0, 0] = acc_ref[0] / (M * D)


@functools.lru_cache(maxsize=None)
def _make_gather(h):
    mesh = plsc.VectorSubcoreMesh(core_axis_name="c", subcore_axis_name="s")
    out_type = (jax.ShapeDtypeStruct((BSZ * T, C), jnp.float32)
                if h == 0 else ())

    @functools.partial(
        pl.kernel, mesh=mesh,
        out_type=out_type,
        scratch_types=[
            pltpu.VMEM((RW,), jnp.int32),
            pltpu.VMEM((CH, C), jnp.float32),
            pltpu.VMEM((CH, C), jnp.float32),
            pltpu.VMEM((CH, C), jnp.float32),
            pltpu.SemaphoreType.DMA,
            pltpu.SemaphoreType.DMA,
            pltpu.SemaphoreType.DMA,
            pltpu.SemaphoreType.DMA,
            pltpu.SemaphoreType.DMA,
            pltpu.SemaphoreType.DMA,
        ],
    )
    def gather(lut_hbm, idx2_hbm, out_hbm, idx_v, rows_a, rows_b, rows_c,
               gs_a, gs_b, gs_c, ss_a, ss_b, ss_c):
        wid = lax.axis_index("s") * NC + lax.axis_index("c")
        src = wid * RW
        pltpu.sync_copy(idx2_hbm.at[pl.ds(src, RW)], idx_v)
        bufs = (rows_a, rows_b, rows_c)
        gsem = (gs_a, gs_b, gs_c)
        ssem = (ss_a, ss_b, ss_c)
        nst = RW // CH
        nbuf = 3

        def gstart(j):
            return pltpu.async_copy(
                lut_hbm.at[idx_v.at[pl.ds(j * CH, CH)]], bufs[j % nbuf],
                gsem[j % nbuf])

        def sstart(j):
            dst = h * DS * MH + src + j * CH
            return pltpu.async_copy(
                bufs[j % nbuf], out_hbm.at[pl.ds(dst, CH)], ssem[j % nbuf])

        gops = [gstart(j) for j in range(min(nbuf, nst))]
        sops = [None] * nst
        for j in range(nst):
            gops[j % nbuf].wait()
            sops[j] = sstart(j)
            if j + nbuf < nst:
                sops[j].wait()
                gops[j % nbuf] = gstart(j + nbuf)
        for j in range(max(0, nst - nbuf), nst):
            sops[j].wait()

    return gather


def _dist_call(h, x, wf, b_in2, embed, en):
    blocks_per_chunk = MH // BM
    bpb = N // BM

    def x_map(c):
        def m(i, h=h, c=c):
            g = h * blocks_per_chunk + i
            return (g // bpb, g % bpb, c)
        return m

    x_specs = [pl.BlockSpec((1, DS * BM, C // NCCH), x_map(c))
               for c in range(NCCH)]
    return pl.pallas_call(
        _dist_body,
        grid=(blocks_per_chunk,),
        in_specs=x_specs + [
            pl.BlockSpec((D, DS * C), lambda i: (0, 0)),
            pl.BlockSpec((1, D), lambda i: (0, 0)),
            pl.BlockSpec((K, D), lambda i: (0, 0)),
            pl.BlockSpec((K,), lambda i: (0,)),
        ],
        out_specs=[
            pl.BlockSpec((BM,), lambda i: (i,)),
            pl.BlockSpec((1, 1), lambda i: (0, 0),
                         memory_space=pltpu.SMEM),
        ],
        out_shape=[
            jax.ShapeDtypeStruct((MH,), jnp.int32),
            jax.ShapeDtypeStruct((1, 1), jnp.float32),
        ],
        scratch_shapes=[pltpu.SMEM((1,), jnp.float32)],
    )(*([x] * NCCH), wf, b_in2, embed, en)


def kernel(x, W_in, b_in, embed, W_out, b_out):
    wf = W_in.transpose(0, 2, 1).reshape(D, DS * C)
    wout = W_out[:, :, 0]
    b_in2 = b_in.reshape(1, D)

    lut, en = pl.pallas_call(
        _lut_body,
        grid=(K // BK,),
        in_specs=[
            pl.BlockSpec((BK, D), lambda i: (i, 0)),
            pl.BlockSpec((C, D), lambda i: (0, 0)),
            pl.BlockSpec((1, C), lambda i: (0, 0)),
        ],
        out_specs=[
            pl.BlockSpec((BK, C), lambda i: (i, 0)),
            pl.BlockSpec((BK,), lambda i: (i,)),
        ],
        out_shape=[
            jax.ShapeDtypeStruct((K, C), jnp.float32),
            jax.ShapeDtypeStruct((K,), jnp.float32),
        ],
    )(embed, wout, b_out.reshape(1, C))

    loss = jnp.zeros((), jnp.float32)
    out_ref = None
    for h in range(H):
        idx_h, part_h = _dist_call(h, x, wf, b_in2, embed, en)
        idx2_h = jnp.repeat(idx_h, DS)
        if h == 0:
            out0 = _make_gather(0)(lut, idx2_h)
            out_ref = jax.new_ref(out0)
        else:
            _make_gather(h)(lut, idx2_h, out_ref)
        loss = loss + part_h.reshape(())

    return (out_ref[...].reshape(BSZ, T, C), loss)


# speedup vs baseline: 1.0145x; 1.0145x over previous
import functools

import jax
import jax.numpy as jnp
from jax import lax
from jax.experimental import pallas as pl
from jax.experimental.pallas import tpu as pltpu
from jax.experimental.pallas import tpu_sc as plsc

BSZ, T, C = 4, 4096, 1024
D = 1024
K = 2048
DS = 2
N = T // DS
M = BSZ * N

H = 4
MH = M // H
BM = 512
BK = 512

NCCH = 8
NC = 2
NS = 16
NW = NC * NS
RW = DS * MH // NW
CH = 64


def _lut_body(embed_ref, wout_ref, bout_ref, lut_ref, en_ref):
    e = embed_ref[:]
    lut_ref[:] = lax.dot_general(
        e, wout_ref[:], (((1,), (1,)), ((), ())),
        preferred_element_type=jnp.float32) + bout_ref[:]
    en_ref[:] = jnp.sum(e * e, axis=1)


def _dist_body(*refs):
    (x_refs, (wf_ref, bin_ref, embed_ref, en_ref),
     (idx_ref, loss_ref), (acc_ref,)) = (refs[:NCCH], refs[NCCH:NCCH + 4],
                                         refs[NCCH + 4:NCCH + 6],
                                         refs[NCCH + 6:])
    i = pl.program_id(0)
    xw = jnp.concatenate(
        [r[0, 0::2, :] for r in x_refs] + [r[0, 1::2, :] for r in x_refs],
        axis=1)
    z = lax.dot_general(
        xw, wf_ref[:], (((1,), (1,)), ((), ())),
        preferred_element_type=jnp.float32) + bin_ref[:]
    s = lax.dot_general(
        z, embed_ref[:], (((1,), (1,)), ((), ())),
        preferred_element_type=jnp.float32)
    zn = jnp.sum(z * z, axis=1, keepdims=True)
    dist = (zn - 2.0 * s) + en_ref[:][None, :]
    idx_ref[:] = jnp.argmin(dist, axis=1).astype(jnp.int32)
    mind = jnp.min(dist, axis=1)

    @pl.when(i == 0)
    def _():
        acc_ref[0] = 0.0

    acc_ref[0] += jnp.sum(mind)

    @pl.when(i == pl.num_programs(0) - 1)
    def _():
        loss_ref[0, 0] = acc_ref[0] / (M * D)


@functools.lru_cache(maxsize=None)
def _make_gather(h):
    mesh = plsc.VectorSubcoreMesh(core_axis_name="c", subcore_axis_name="s")
    out_type = (jax.ShapeDtypeStruct((BSZ * T, C), jnp.float32)
                if h == 0 else ())

    @functools.partial(
        pl.kernel, mesh=mesh,
        out_type=out_type,
        scratch_types=[
            pltpu.VMEM((CH,), jnp.int32),
            pltpu.VMEM((CH, C), jnp.float32),
            pltpu.SemaphoreType.DMA,
        ],
    )
    def gather(lut_hbm, idx2_hbm, out_hbm, idx_v, rows_v, sem):
        wid = lax.axis_index("s") * NC + lax.axis_index("c")
        src = wid * RW

        def step(j, carry):
            off = src + j * CH
            pltpu.sync_copy(idx2_hbm.at[pl.ds(off, CH)], idx_v)
            pltpu.async_copy(lut_hbm.at[idx_v], rows_v, sem).wait()
            pltpu.sync_copy(rows_v, out_hbm.at[pl.ds(h * DS * MH + off, CH)])
            return carry

        lax.fori_loop(0, RW // CH, step, 0)

    return gather


def _dist_call(h, x, wf, b_in2, embed, en):
    blocks_per_chunk = MH // BM
    bpb = N // BM

    def x_map(c):
        def m(i, h=h, c=c):
            g = h * blocks_per_chunk + i
            return (g // bpb, g % bpb, c)
        return m

    x_specs = [pl.BlockSpec((1, DS * BM, C // NCCH), x_map(c))
               for c in range(NCCH)]
    return pl.pallas_call(
        _dist_body,
        grid=(blocks_per_chunk,),
        in_specs=x_specs + [
            pl.BlockSpec((D, DS * C), lambda i: (0, 0)),
            pl.BlockSpec((1, D), lambda i: (0, 0)),
            pl.BlockSpec((K, D), lambda i: (0, 0)),
            pl.BlockSpec((K,), lambda i: (0,)),
        ],
        out_specs=[
            pl.BlockSpec((BM,), lambda i: (i,)),
            pl.BlockSpec((1, 1), lambda i: (0, 0),
                         memory_space=pltpu.SMEM),
        ],
        out_shape=[
            jax.ShapeDtypeStruct((MH,), jnp.int32),
            jax.ShapeDtypeStruct((1, 1), jnp.float32),
        ],
        scratch_shapes=[pltpu.SMEM((1,), jnp.float32)],
    )(*([x] * NCCH), wf, b_in2, embed, en)


def kernel(x, W_in, b_in, embed, W_out, b_out):
    wf = W_in.transpose(0, 2, 1).reshape(D, DS * C)
    wout = W_out[:, :, 0]
    b_in2 = b_in.reshape(1, D)

    lut, en = pl.pallas_call(
        _lut_body,
        grid=(K // BK,),
        in_specs=[
            pl.BlockSpec((BK, D), lambda i: (i, 0)),
            pl.BlockSpec((C, D), lambda i: (0, 0)),
            pl.BlockSpec((1, C), lambda i: (0, 0)),
        ],
        out_specs=[
            pl.BlockSpec((BK, C), lambda i: (i, 0)),
            pl.BlockSpec((BK,), lambda i: (i,)),
        ],
        out_shape=[
            jax.ShapeDtypeStruct((K, C), jnp.float32),
            jax.ShapeDtypeStruct((K,), jnp.float32),
        ],
    )(embed, wout, b_out.reshape(1, C))

    loss = jnp.zeros((), jnp.float32)
    out_ref = None
    for h in range(H):
        idx_h, part_h = _dist_call(h, x, wf, b_in2, embed, en)
        idx2_h = jnp.repeat(idx_h, DS)
        if h == 0:
            out0 = _make_gather(0)(lut, idx2_h)
            out_ref = jax.new_ref(out0)
        else:
            _make_gather(h)(lut, idx2_h, out_ref)
        loss = loss + part_h.reshape(())

    return (out_ref[...].reshape(BSZ, T, C), loss)
